# Initial kernel scaffold; baseline (speedup 1.0000x reference)
import jax
import jax.numpy as jnp
from jax.experimental import pallas as pl


def kernel(x, edge_index, batch, W_in, b_in, g_in, be_in, W_mid, b_mid, g_mid, be_mid, W_out, b_out, g_out, be_out, fc1_W, fc1_b, fc2_W, fc2_b):
    raise NotImplementedError("write your pallas kernel here")


# speedup vs baseline: 6.7251x; 6.7251x over previous
import functools

import jax
import jax.numpy as jnp
from jax import lax
from jax.experimental import pallas as pl
from jax.experimental.pallas import tpu as pltpu
from jax.experimental.pallas import tpu_sc as plsc

_N = 10000
_E = 320000
_DH = 64
_B = 64
_NUM_LAYERS = 8

_NC = 2
_NS = 16
_NW = _NC * _NS
_CHUNK = 128
_EPW = -(-_E // _NW)
_NCH = -(-_EPW // _CHUNK)
_EPAD = _NW * _NCH * _CHUNK
_NACC = 10112
_RPS = _NACC // _NS

_DW = 128
_ROWBLK = 1000
_NBLK = _N // _ROWBLK


def _prop_body(m_hbm, srcs_hbm, dsts_hbm, zeros_hbm, out0, out1,
               src_v, dst_v, rows_v, acc, sem):
    c = lax.axis_index("c")
    s = lax.axis_index("s")
    w = s * _NC + c

    pltpu.sync_copy(srcs_hbm.at[w], src_v)
    pltpu.sync_copy(dsts_hbm.at[w], dst_v)
    pltpu.sync_copy(zeros_hbm.at[pl.ds(s * _RPS, _RPS)],
                    acc.at[pl.ds(s * _RPS, _RPS)])
    plsc.subcore_barrier()

    def chunk(j, carry):
        pltpu.async_copy(m_hbm.at[src_v.at[j]], rows_v, sem).wait()
        pltpu.sync_copy(rows_v, acc.at[dst_v.at[j]], add=True)
        return carry

    lax.fori_loop(0, _NCH, chunk, 0)
    plsc.subcore_barrier()

    @pl.when(c == 0)
    def _():
        pltpu.sync_copy(acc.at[pl.ds(s * _RPS, _RPS)],
                        out0.at[pl.ds(s * _RPS, _RPS)])

    @pl.when(c == 1)
    def _():
        pltpu.sync_copy(acc.at[pl.ds(s * _RPS, _RPS)],
                        out1.at[pl.ds(s * _RPS, _RPS)])


@functools.cache
def _get_propagate():
    return pl.kernel(
        _prop_body,
        out_type=(jax.ShapeDtypeStruct((_NACC, _DW), jnp.float32),
                  jax.ShapeDtypeStruct((_NACC, _DW), jnp.float32)),
        mesh=plsc.VectorSubcoreMesh(core_axis_name="c", subcore_axis_name="s",
                                    num_cores=_NC, num_subcores=_NS),
        scratch_types=[
            pltpu.VMEM((_NCH, _CHUNK), jnp.int32),
            pltpu.VMEM((_NCH, _CHUNK), jnp.int32),
            pltpu.VMEM((_CHUNK, _DW), jnp.float32),
            pltpu.VMEM_SHARED((_NACC, _DW), jnp.float32),
            pltpu.SemaphoreType.DMA,
        ],
    )


def _dot(a, b):
    return lax.dot_general(a, b, (((1,), (0,)), ((), ())),
                           precision=lax.Precision.HIGHEST,
                           preferred_element_type=jnp.float32)


def _first_body(x_ref, w_ref, d0_ref, d1_ref, m_ref, dinv_ref):
    dinv = lax.rsqrt(1.0 + d0_ref[...][:, :1] + d1_ref[...][:, :1])
    mv = _dot(x_ref[...], w_ref[...]) * dinv
    m_ref[...] = jnp.concatenate([mv, jnp.zeros_like(mv)], axis=1)
    dinv_ref[...] = dinv


def _first_call(x, w, d0, d1):
    d_in = x.shape[1]
    return pl.pallas_call(
        _first_body,
        grid=(_NBLK,),
        in_specs=[
            pl.BlockSpec((_ROWBLK, d_in), lambda i: (i, 0)),
            pl.BlockSpec((d_in, _DH), lambda i: (0, 0)),
            pl.BlockSpec((_ROWBLK, _DW), lambda i: (i, 0)),
            pl.BlockSpec((_ROWBLK, _DW), lambda i: (i, 0)),
        ],
        out_specs=[
            pl.BlockSpec((_ROWBLK, _DW), lambda i: (i, 0)),
            pl.BlockSpec((_ROWBLK, 1), lambda i: (i, 0)),
        ],
        out_shape=[
            jax.ShapeDtypeStruct((_NACC, _DW), jnp.float32),
            jax.ShapeDtypeStruct((_N, 1), jnp.float32),
        ],
    )(x, w, d0, d1)


def _r64(ref):
    v = ref[...]
    return v[:, :_DH]


def _layer_body(p0_ref, p1_ref, m_ref, hp_ref, dinv_ref, b_ref, g_ref,
                be_ref, wn_ref, h_ref, mn_ref, *, use_res):
    dinv = dinv_ref[...]
    a = dinv * (_r64(p0_ref) + _r64(p1_ref) + _r64(m_ref)) + b_ref[...]
    mu = jnp.mean(a, axis=-1, keepdims=True)
    var = jnp.mean((a - mu) ** 2, axis=-1, keepdims=True)
    hn = (a - mu) * lax.rsqrt(var + 1e-5) * g_ref[...] + be_ref[...]
    h = jnp.maximum(hn, 0.0)
    if use_res:
        h = h + _r64(hp_ref)
    h_ref[...] = h
    mn = _dot(h, wn_ref[...]) * dinv
    mn_ref[...] = jnp.concatenate([mn, jnp.zeros_like(mn)], axis=1)


def _layer_call(p0, p1, m, hp, dinv, b, g, be, wn, use_res):
    return pl.pallas_call(
        functools.partial(_layer_body, use_res=use_res),
        grid=(_NBLK,),
        in_specs=[
            pl.BlockSpec((_ROWBLK, _DW), lambda i: (i, 0)),
            pl.BlockSpec((_ROWBLK, _DW), lambda i: (i, 0)),
            pl.BlockSpec((_ROWBLK, _DW), lambda i: (i, 0)),
            pl.BlockSpec((_ROWBLK, hp.shape[1]), lambda i: (i, 0)),
            pl.BlockSpec((_ROWBLK, 1), lambda i: (i, 0)),
            pl.BlockSpec((1, _DH), lambda i: (0, 0)),
            pl.BlockSpec((1, _DH), lambda i: (0, 0)),
            pl.BlockSpec((1, _DH), lambda i: (0, 0)),
            pl.BlockSpec((_DH, _DH), lambda i: (0, 0)),
        ],
        out_specs=[
            pl.BlockSpec((_ROWBLK, _DH), lambda i: (i, 0)),
            pl.BlockSpec((_ROWBLK, _DW), lambda i: (i, 0)),
        ],
        out_shape=[
            jax.ShapeDtypeStruct((_N, _DH), jnp.float32),
            jax.ShapeDtypeStruct((_NACC, _DW), jnp.float32),
        ],
    )(p0, p1, m, hp, dinv, b, g, be, wn)


def _pool_body(h_ref, batch_ref, fc1w_ref, fc1b_ref, fc2w_ref, fc2b_ref,
               out_ref, sums, counts):
    i = pl.program_id(0)

    @pl.when(i == 0)
    def _():
        sums[...] = jnp.zeros_like(sums)
        counts[...] = jnp.zeros_like(counts)

    gid = lax.broadcasted_iota(jnp.int32, (1, _B), 1)
    onehot = (batch_ref[...] == gid).astype(jnp.float32)
    sums[...] += lax.dot_general(onehot, h_ref[...],
                                 (((0,), (0,)), ((), ())),
                                 precision=lax.Precision.HIGHEST,
                                 preferred_element_type=jnp.float32)
    ones_col = jnp.ones((_ROWBLK, 1), jnp.float32)
    counts[...] += lax.dot_general(onehot, ones_col,
                                   (((0,), (0,)), ((), ())),
                                   precision=lax.Precision.HIGHEST,
                                   preferred_element_type=jnp.float32)

    @pl.when(i == _NBLK - 1)
    def _():
        pooled = sums[...] / jnp.maximum(counts[...], 1.0)
        z = jnp.maximum(_dot(pooled, fc1w_ref[...]) + fc1b_ref[...], 0.0)
        out_ref[...] = _dot(z, fc2w_ref[...]) + fc2b_ref[...]


def _pool_call(h, batch2d, fc1w, fc1b, fc2w, fc2b):
    return pl.pallas_call(
        _pool_body,
        grid=(_NBLK,),
        in_specs=[
            pl.BlockSpec((_ROWBLK, _DH), lambda i: (i, 0)),
            pl.BlockSpec((_ROWBLK, 1), lambda i: (i, 0)),
            pl.BlockSpec((_DH, _DH), lambda i: (0, 0)),
            pl.BlockSpec((1, _DH), lambda i: (0, 0)),
            pl.BlockSpec((_DH, 1), lambda i: (0, 0)),
            pl.BlockSpec((1, 1), lambda i: (0, 0)),
        ],
        out_specs=pl.BlockSpec((_B, 1), lambda i: (0, 0)),
        out_shape=jax.ShapeDtypeStruct((_B, 1), jnp.float32),
        scratch_shapes=[
            pltpu.VMEM((_B, _DH), jnp.float32),
            pltpu.VMEM((_B, 1), jnp.float32),
        ],
    )(h, batch2d, fc1w, fc1b, fc2w, fc2b)


def kernel(x, edge_index, batch, W_in, b_in, g_in, be_in, W_mid, b_mid,
           g_mid, be_mid, W_out, b_out, g_out, be_out, fc1_W, fc1_b,
           fc2_W, fc2_b):
    src, dst = edge_index[0], edge_index[1]
    pad = _EPAD - _E
    srcs = jnp.concatenate(
        [src, jnp.zeros((pad,), src.dtype)]).reshape(_NW, _NCH, _CHUNK)
    dsts = jnp.concatenate(
        [dst, jnp.full((pad,), _N, dst.dtype)]).reshape(_NW, _NCH, _CHUNK)
    zeros_acc = jnp.zeros((_NACC, _DW), jnp.float32)
    ones_n = jnp.ones((_NACC, _DW), jnp.float32)

    d0, d1 = _get_propagate()(ones_n, srcs, dsts, zeros_acc)

    m, dinv = _first_call(x, W_in, d0, d1)

    params = ([(b_in, g_in, be_in)]
              + [(b_mid, g_mid, be_mid)] * (_NUM_LAYERS - 2)
              + [(b_out, g_out, be_out)])
    h = m
    for l in range(_NUM_LAYERS):
        p0, p1 = _get_propagate()(m, srcs, dsts, zeros_acc)
        b, g, be = params[l]
        wn = W_out if l == _NUM_LAYERS - 2 else W_mid
        use_res = l in (2, 4, 6)
        h_new, m_new = _layer_call(p0, p1, m, h, dinv,
                                   b.reshape(1, _DH), g.reshape(1, _DH),
                                   be.reshape(1, _DH), wn, use_res)
        h, m = h_new, m_new

    return _pool_call(h, batch.reshape(_N, 1), fc1_W,
                      fc1_b.reshape(1, _DH), fc2_W, fc2_b.reshape(1, 1))
